# baseline (speedup 1.0000x reference)
import functools

import jax
import jax.numpy as jnp
from jax import lax
from jax.experimental import pallas as pl
from jax.experimental.pallas import tpu as pltpu
from jax.experimental.pallas import tpu_sc as plsc

VOCAB = 8192
D = 8192
DH = D // 2
NC = 2
NS = 16
NW = NC * NS
K = 8


def _make_gather(n):
    pw = n // (NW // 2)
    chunks = pw // K
    pairs = chunks // 2
    mesh = plsc.VectorSubcoreMesh(core_axis_name="c", subcore_axis_name="s")

    @functools.partial(
        pl.kernel,
        mesh=mesh,
        out_type=jax.ShapeDtypeStruct((n, D), jnp.float32),
        scratch_types=[
            pltpu.VMEM((pw,), jnp.int32),
            [pltpu.VMEM((K, DH), jnp.float32)] * 2,
            [pltpu.SemaphoreType.DMA] * 2,
            [pltpu.SemaphoreType.DMA] * 2,
        ],
    )
    def gather_kernel(table_hbm, idx_hbm, out_hbm, idx_v, rows_v, gsems, ssems):
        cid = lax.axis_index("c")
        sid = lax.axis_index("s")
        wid = sid * NC + cid
        rw = wid // 2
        col0 = (wid % 2) * DH
        base = rw * pw
        t_per = idx_hbm.shape[1]
        pltpu.sync_copy(
            idx_hbm.at[base // t_per, pl.ds(base % t_per, pw)], idx_v
        )

        def g_copy(g, c):
            return pltpu.make_async_copy(
                table_hbm.at[idx_v.at[pl.ds(c * K, K)], pl.ds(col0, DH)],
                rows_v[g],
                gsems[g],
            )

        def s_copy(g, c):
            return pltpu.make_async_copy(
                rows_v[g],
                out_hbm.at[pl.ds(base + c * K, K), pl.ds(col0, DH)],
                ssems[g],
            )

        g_copy(0, 0).start()

        def pair_body(r2, carry):
            ca = 2 * r2
            cb = ca + 1
            g_copy(0, ca).wait()
            s_copy(0, ca).start()
            @pl.when(r2 > 0)
            def _():
                s_copy(1, cb - 2).wait()
            g_copy(1, cb).start()
            g_copy(1, cb).wait()
            s_copy(1, cb).start()
            s_copy(0, ca).wait()
            @pl.when(r2 < pairs - 1)
            def _():
                g_copy(0, ca + 2).start()
            return carry

        lax.fori_loop(0, pairs, pair_body, 0)

        s_copy(1, 2 * pairs - 1).wait()

    return gather_kernel


def kernel(idx, logits_table):
    b, t = idx.shape
    n = b * t
    out2 = _make_gather(n)(logits_table, idx.astype(jnp.int32))
    return out2.reshape(b, t, D)
